# Initial kernel scaffold; baseline (speedup 1.0000x reference)
import functools

import jax
import jax.numpy as jnp
from jax import lax
from jax.experimental import pallas as pl
from jax.experimental.pallas import tpu as pltpu
from jax.experimental.pallas import tpu_sc as plsc

K = 100000
D = 32
B = 1024

NC = 2
NS = 16
NW = NC * NS
KPAD = 102400
ROWS_W = KPAD // NW
CHUNK = 128
NCHUNK = ROWS_W // CHUNK

TK = 2048
NT = KPAD // TK


def _sc_added_body(w_hbm, p1_hbm, p2_hbm, out_hbm,
                   idx1_v, idx2_v, acc_v, r1_v, r2_v, sem1, sem2):
    wid = lax.axis_index("s") * NC + lax.axis_index("c")
    base_row = wid * ROWS_W

    def chunk_body(ci, carry):
        start = base_row + ci * CHUNK
        pltpu.sync_copy(p1_hbm.at[pl.ds(start, CHUNK)], idx1_v)
        pltpu.sync_copy(p2_hbm.at[pl.ds(start, CHUNK)], idx2_v)
        cp1 = pltpu.async_copy(w_hbm.at[idx1_v], r1_v, sem1)
        cp2 = pltpu.async_copy(w_hbm.at[idx2_v], r2_v, sem2)
        pltpu.sync_copy(w_hbm.at[pl.ds(start, CHUNK)], acc_v)
        cp1.wait()
        cp2.wait()

        def row_body(r, c2):
            lo = pl.ds(0, 16)
            hi = pl.ds(16, 16)
            acc_v[r, lo] = acc_v[r, lo] + r1_v[r, lo] + r2_v[r, lo]
            acc_v[r, hi] = acc_v[r, hi] + r1_v[r, hi] + r2_v[r, hi]
            return c2

        lax.fori_loop(0, CHUNK, row_body, 0)
        pltpu.sync_copy(acc_v, out_hbm.at[pl.ds(start, CHUNK)])
        return carry

    lax.fori_loop(0, NCHUNK, chunk_body, 0)


_sc_added = functools.partial(
    pl.kernel,
    mesh=plsc.VectorSubcoreMesh(core_axis_name="c", subcore_axis_name="s"),
    out_type=jax.ShapeDtypeStruct((KPAD, D), jnp.float32),
    scratch_types=[
        pltpu.VMEM((CHUNK,), jnp.int32),
        pltpu.VMEM((CHUNK,), jnp.int32),
        pltpu.VMEM((CHUNK, D), jnp.float32),
        pltpu.VMEM((CHUNK, D), jnp.float32),
        pltpu.VMEM((CHUNK, D), jnp.float32),
        pltpu.SemaphoreType.DMA,
        pltpu.SemaphoreType.DMA,
    ],
)(_sc_added_body)


def _tc_body(x_ref, y_ref, aw_ref, logits_ref, loss_ref, m_ref, s_ref, ly_ref):
    k = pl.program_id(0)

    @pl.when(k == 0)
    def _init():
        m_ref[...] = jnp.full((B, 1), -jnp.inf, jnp.float32)
        s_ref[...] = jnp.zeros((B, 1), jnp.float32)
        ly_ref[...] = jnp.zeros((B, 1), jnp.float32)

    logits = lax.dot_general(
        x_ref[...], aw_ref[...],
        dimension_numbers=(((1,), (1,)), ((), ())),
        preferred_element_type=jnp.float32,
    )
    logits_ref[...] = logits

    cols = k * TK + lax.broadcasted_iota(jnp.int32, (1, TK), 1)
    masked = jnp.where(cols < K, logits, -jnp.inf)
    tile_max = jnp.max(masked, axis=1, keepdims=True)
    m_old = m_ref[...]
    m_new = jnp.maximum(m_old, tile_max)
    p = jnp.exp(masked - m_new)
    s_ref[...] = s_ref[...] * jnp.exp(m_old - m_new) + jnp.sum(p, axis=1, keepdims=True)
    m_ref[...] = m_new

    ymask = cols == y_ref[...]
    ly_ref[...] += jnp.sum(jnp.where(ymask, logits, 0.0), axis=1, keepdims=True)

    @pl.when(k == NT - 1)
    def _fin():
        lse = m_ref[...] + jnp.log(s_ref[...])
        nll = lse - ly_ref[...]
        loss_ref[...] = jnp.sum(nll, axis=0, keepdims=True) / B


_tc_call = pl.pallas_call(
    _tc_body,
    grid=(NT,),
    in_specs=[
        pl.BlockSpec((B, D), lambda k: (0, 0)),
        pl.BlockSpec((B, 1), lambda k: (0, 0)),
        pl.BlockSpec((TK, D), lambda k: (k, 0)),
    ],
    out_specs=[
        pl.BlockSpec((B, TK), lambda k: (0, k)),
        pl.BlockSpec((1, 1), lambda k: (0, 0)),
    ],
    out_shape=[
        jax.ShapeDtypeStruct((B, K), jnp.float32),
        jax.ShapeDtypeStruct((1, 1), jnp.float32),
    ],
    scratch_shapes=[
        pltpu.VMEM((B, 1), jnp.float32),
        pltpu.VMEM((B, 1), jnp.float32),
        pltpu.VMEM((B, 1), jnp.float32),
    ],
    compiler_params=pltpu.CompilerParams(
        dimension_semantics=("arbitrary",),
    ),
)


def kernel(weights, x, y, path_idx):
    pad = KPAD - K
    p1 = jnp.concatenate([path_idx[:, 0], jnp.zeros((pad,), jnp.int32)])
    p2 = jnp.concatenate([path_idx[:, 1], jnp.zeros((pad,), jnp.int32)])
    added_pad = _sc_added(weights, p1, p2)

    y2d = y.reshape(B, 1).astype(jnp.int32)
    logits, loss = _tc_call(x, y2d, added_pad)
    return (loss[0, 0], logits, added_pad[:K])


# speedup vs baseline: 1.1622x; 1.1622x over previous
import functools

import jax
import jax.numpy as jnp
from jax import lax
from jax.experimental import pallas as pl
from jax.experimental.pallas import tpu as pltpu
from jax.experimental.pallas import tpu_sc as plsc

K = 100000
D = 32
B = 1024

NC = 2
NS = 16
NW = NC * NS
KPAD = 100352
ROWS_W = KPAD // NW
CHUNK = 112
NCHUNK = ROWS_W // CHUNK

TK = 2048
NT = KPAD // TK


def _sc_added_body(w_hbm, p1_hbm, p2_hbm, out_hbm,
                   idx1_v, idx2_v, acc_v, r1_v, r2_v, sem1, sem2):
    wid = lax.axis_index("s") * NC + lax.axis_index("c")
    base_row = wid * ROWS_W

    def chunk_body(ci, carry):
        start = base_row + ci * CHUNK
        pltpu.sync_copy(p1_hbm.at[pl.ds(start, CHUNK)], idx1_v)
        pltpu.sync_copy(p2_hbm.at[pl.ds(start, CHUNK)], idx2_v)
        cp1 = pltpu.async_copy(w_hbm.at[idx1_v], r1_v, sem1)
        cp2 = pltpu.async_copy(w_hbm.at[idx2_v], r2_v, sem2)
        pltpu.sync_copy(w_hbm.at[pl.ds(start, CHUNK)], acc_v)
        cp1.wait()
        cp2.wait()

        def row_body(r, c2):
            lo = pl.ds(0, 16)
            hi = pl.ds(16, 16)
            acc_v[r, lo] = acc_v[r, lo] + r1_v[r, lo] + r2_v[r, lo]
            acc_v[r, hi] = acc_v[r, hi] + r1_v[r, hi] + r2_v[r, hi]
            return c2

        lax.fori_loop(0, CHUNK, row_body, 0)
        pltpu.sync_copy(acc_v, out_hbm.at[pl.ds(start, CHUNK)])
        return carry

    lax.fori_loop(0, NCHUNK, chunk_body, 0)


@functools.cache
def _sc_added_call():
    return functools.partial(
        pl.kernel,
        mesh=plsc.VectorSubcoreMesh(core_axis_name="c", subcore_axis_name="s"),
        out_type=jax.ShapeDtypeStruct((KPAD, D), jnp.float32),
        scratch_types=[
            pltpu.VMEM((CHUNK,), jnp.int32),
            pltpu.VMEM((CHUNK,), jnp.int32),
            pltpu.VMEM((CHUNK, D), jnp.float32),
            pltpu.VMEM((CHUNK, D), jnp.float32),
            pltpu.VMEM((CHUNK, D), jnp.float32),
            pltpu.SemaphoreType.DMA,
            pltpu.SemaphoreType.DMA,
        ],
        compiler_params=pltpu.CompilerParams(use_tc_tiling_on_sc=False),
    )(_sc_added_body)


def _tc_body(x_ref, y_ref, aw_ref, logits_ref, loss_ref, m_ref, s_ref, ly_ref):
    k = pl.program_id(0)

    @pl.when(k == 0)
    def _init():
        m_ref[...] = jnp.full((B, 1), -jnp.inf, jnp.float32)
        s_ref[...] = jnp.zeros((B, 1), jnp.float32)
        ly_ref[...] = jnp.zeros((B, 1), jnp.float32)

    logits = lax.dot_general(
        x_ref[...], aw_ref[...],
        dimension_numbers=(((1,), (1,)), ((), ())),
        preferred_element_type=jnp.float32,
    )
    logits_ref[...] = logits

    cols = k * TK + lax.broadcasted_iota(jnp.int32, (1, TK), 1)
    masked = jnp.where(cols < K, logits, -jnp.inf)
    tile_max = jnp.max(masked, axis=1, keepdims=True)
    m_old = m_ref[...]
    m_new = jnp.maximum(m_old, tile_max)
    p = jnp.exp(masked - m_new)
    s_ref[...] = s_ref[...] * jnp.exp(m_old - m_new) + jnp.sum(p, axis=1, keepdims=True)
    m_ref[...] = m_new

    ymask = cols == y_ref[...]
    ly_ref[...] += jnp.sum(jnp.where(ymask, logits, 0.0), axis=1, keepdims=True)

    @pl.when(k == NT - 1)
    def _fin():
        lse = m_ref[...] + jnp.log(s_ref[...])
        nll = lse - ly_ref[...]
        loss_ref[...] = jnp.sum(nll, axis=0, keepdims=True) / B


_tc_call = pl.pallas_call(
    _tc_body,
    grid=(NT,),
    in_specs=[
        pl.BlockSpec((B, D), lambda k: (0, 0)),
        pl.BlockSpec((B, 1), lambda k: (0, 0)),
        pl.BlockSpec((TK, D), lambda k: (k, 0)),
    ],
    out_specs=[
        pl.BlockSpec((B, TK), lambda k: (0, k)),
        pl.BlockSpec((1, 1), lambda k: (0, 0)),
    ],
    out_shape=[
        jax.ShapeDtypeStruct((B, K), jnp.float32),
        jax.ShapeDtypeStruct((1, 1), jnp.float32),
    ],
    scratch_shapes=[
        pltpu.VMEM((B, 1), jnp.float32),
        pltpu.VMEM((B, 1), jnp.float32),
        pltpu.VMEM((B, 1), jnp.float32),
    ],
    compiler_params=pltpu.CompilerParams(
        dimension_semantics=("arbitrary",),
    ),
)


def kernel(weights, x, y, path_idx):
    pad = KPAD - K
    p1 = jnp.concatenate([path_idx[:, 0], jnp.zeros((pad,), jnp.int32)])
    p2 = jnp.concatenate([path_idx[:, 1], jnp.zeros((pad,), jnp.int32)])
    added_pad = _sc_added_call()(weights, p1, p2)

    y2d = y.reshape(B, 1).astype(jnp.int32)
    logits, loss = _tc_call(x, y2d, added_pad)
    return (loss[0, 0], logits, added_pad[:K])
